# baseline (speedup 1.0000x reference)
import functools

import jax
import jax.numpy as jnp
from jax import lax
from jax.experimental import pallas as pl
from jax.experimental.pallas import tpu as pltpu
from jax.experimental.pallas import tpu_sc as plsc

_B, _C = 16384, 1000
_NW = 32
_RPW = _B // _NW
_CH = 32
_NCH = _RPW // _CH


def kernel(input, masked_values):
    mesh = plsc.VectorSubcoreMesh(core_axis_name="c", subcore_axis_name="s")

    @functools.partial(
        pl.kernel,
        mesh=mesh,
        compiler_params=pltpu.CompilerParams(
            use_tc_tiling_on_sc=False, needs_layout_passes=False
        ),
        out_type=jax.ShapeDtypeStruct((_B, _C), jnp.float32),
        scratch_types=[
            pltpu.VMEM((_RPW,), jnp.int32),
            pltpu.VMEM((_CH, _C), jnp.float32),
        ],
    )
    def sc(in_hbm, mv_hbm, out_hbm, mv_v, buf_v):
        wid = lax.axis_index("s") * 2 + lax.axis_index("c")
        base = wid * _RPW
        pltpu.sync_copy(mv_hbm.at[pl.ds(base, _RPW)], mv_v)
        neg1 = jnp.full((16,), -1.0, jnp.float32)
        for g in range(_NCH):
            r0 = base + g * _CH
            pltpu.sync_copy(in_hbm.at[pl.ds(r0, _CH)], buf_v)
            for k in range(_CH // 16):
                rows = lax.iota(jnp.int32, 16) + (k * 16)
                cols = mv_v[pl.ds(g * _CH + k * 16, 16)]
                plsc.store_scatter(buf_v, [rows, cols], neg1)
            pltpu.sync_copy(buf_v, out_hbm.at[pl.ds(r0, _CH)])

    return sc(input, masked_values)


# speedup vs baseline: 1.0474x; 1.0474x over previous
import functools

import jax
import jax.numpy as jnp
from jax import lax
from jax.experimental import pallas as pl
from jax.experimental.pallas import tpu as pltpu
from jax.experimental.pallas import tpu_sc as plsc

_B, _C = 16384, 1000
_NW = 32
_RPW = _B // _NW
_CH = 32
_NCH = _RPW // _CH
_NBUF = 4
_PF = 3


def kernel(input, masked_values):
    mesh = plsc.VectorSubcoreMesh(core_axis_name="c", subcore_axis_name="s")

    @functools.partial(
        pl.kernel,
        mesh=mesh,
        compiler_params=pltpu.CompilerParams(
            use_tc_tiling_on_sc=False, needs_layout_passes=False
        ),
        out_type=jax.ShapeDtypeStruct((_B, _C), jnp.float32),
        scratch_types=[
            pltpu.VMEM((_RPW,), jnp.int32),
            [pltpu.VMEM((_CH, _C), jnp.float32) for _ in range(_NBUF)],
            [pltpu.SemaphoreType.DMA for _ in range(_NBUF)],
            [pltpu.SemaphoreType.DMA for _ in range(_NBUF)],
        ],
    )
    def sc(in_hbm, mv_hbm, out_hbm, mv_v, bufs, in_sems, out_sems):
        wid = lax.axis_index("s") * 2 + lax.axis_index("c")
        base = wid * _RPW
        pltpu.sync_copy(mv_hbm.at[pl.ds(base, _RPW)], mv_v)
        neg1 = jnp.full((16,), -1.0, jnp.float32)

        cin = [None] * _NBUF
        cout = [None] * _NBUF

        def start_in(g):
            s = g % _NBUF
            cin[s] = pltpu.async_copy(
                in_hbm.at[pl.ds(base + g * _CH, _CH)], bufs[s], in_sems[s]
            )

        for j in range(min(_PF, _NCH)):
            start_in(j)
        for g in range(_NCH):
            s = g % _NBUF
            pf = g + _PF
            if pf < _NCH:
                if pf >= _NBUF:
                    cout[pf % _NBUF].wait()
                start_in(pf)
            cin[s].wait()
            for k in range(_CH // 16):
                rows = lax.iota(jnp.int32, 16) + (k * 16)
                cols = mv_v[pl.ds(g * _CH + k * 16, 16)]
                plsc.store_scatter(bufs[s], [rows, cols], neg1)
            cout[s] = pltpu.async_copy(
                bufs[s], out_hbm.at[pl.ds(base + g * _CH, _CH)], out_sems[s]
            )
        for g in range(_NCH - _NBUF, _NCH):
            cout[g % _NBUF].wait()

    return sc(input, masked_values)


# speedup vs baseline: 1.8032x; 1.7216x over previous
import functools

import jax
import jax.numpy as jnp
from jax import lax
from jax.experimental import pallas as pl
from jax.experimental.pallas import tpu as pltpu
from jax.experimental.pallas import tpu_sc as plsc

_B, _C = 16384, 1000
_NW = 32
_RPW = _B // _NW
_CH = 32
_NCH = _RPW // _CH
_NBUF = 3
_PF = 2


def kernel(input, masked_values):
    mesh = plsc.VectorSubcoreMesh(core_axis_name="c", subcore_axis_name="s")

    @functools.partial(
        pl.kernel,
        mesh=mesh,
        compiler_params=pltpu.CompilerParams(
            use_tc_tiling_on_sc=True, needs_layout_passes=False
        ),
        out_type=jax.ShapeDtypeStruct((_B, _C), jnp.float32),
        scratch_types=[
            pltpu.VMEM((_RPW,), jnp.int32),
            [pltpu.VMEM((_CH, _C), jnp.float32) for _ in range(_NBUF)],
            [pltpu.SemaphoreType.DMA for _ in range(_NBUF)],
            [pltpu.SemaphoreType.DMA for _ in range(_NBUF)],
        ],
    )
    def sc(in_hbm, mv_hbm, out_hbm, mv_v, bufs, in_sems, out_sems):
        wid = lax.axis_index("s") * 2 + lax.axis_index("c")
        base = wid * _RPW
        pltpu.sync_copy(mv_hbm.at[pl.ds(base, _RPW)], mv_v)
        neg1 = jnp.full((16,), -1.0, jnp.float32)

        cin = [None] * _NBUF
        cout = [None] * _NBUF

        def start_in(g):
            s = g % _NBUF
            cin[s] = pltpu.async_copy(
                in_hbm.at[pl.ds(base + g * _CH, _CH)], bufs[s], in_sems[s]
            )

        for j in range(min(_PF, _NCH)):
            start_in(j)
        for g in range(_NCH):
            s = g % _NBUF
            pf = g + _PF
            if pf < _NCH:
                if pf >= _NBUF:
                    cout[pf % _NBUF].wait()
                start_in(pf)
            cin[s].wait()
            for k in range(_CH // 16):
                rows = lax.iota(jnp.int32, 16) + (k * 16)
                cols = mv_v[pl.ds(g * _CH + k * 16, 16)]
                plsc.store_scatter(bufs[s], [rows, cols], neg1)
            cout[s] = pltpu.async_copy(
                bufs[s], out_hbm.at[pl.ds(base + g * _CH, _CH)], out_sems[s]
            )
        for g in range(_NCH - _NBUF, _NCH):
            cout[g % _NBUF].wait()

    return sc(input, masked_values)


# speedup vs baseline: 1.8076x; 1.0024x over previous
import functools

import jax
import jax.numpy as jnp
from jax import lax
from jax.experimental import pallas as pl
from jax.experimental.pallas import tpu as pltpu
from jax.experimental.pallas import tpu_sc as plsc

_B, _C = 16384, 1000
_NW = 32
_RPW = _B // _NW
_CH = 16
_NCH = _RPW // _CH
_NBUF = 7
_PF = 5


def kernel(input, masked_values):
    mesh = plsc.VectorSubcoreMesh(core_axis_name="c", subcore_axis_name="s")

    @functools.partial(
        pl.kernel,
        mesh=mesh,
        compiler_params=pltpu.CompilerParams(
            use_tc_tiling_on_sc=True, needs_layout_passes=False
        ),
        out_type=jax.ShapeDtypeStruct((_B, _C), jnp.float32),
        scratch_types=[
            pltpu.VMEM((_RPW,), jnp.int32),
            [pltpu.VMEM((_CH, _C), jnp.float32) for _ in range(_NBUF)],
            [pltpu.SemaphoreType.DMA for _ in range(_NBUF)],
            [pltpu.SemaphoreType.DMA for _ in range(_NBUF)],
        ],
    )
    def sc(in_hbm, mv_hbm, out_hbm, mv_v, bufs, in_sems, out_sems):
        wid = lax.axis_index("s") * 2 + lax.axis_index("c")
        base = wid * _RPW
        pltpu.sync_copy(mv_hbm.at[pl.ds(base, _RPW)], mv_v)
        neg1 = jnp.full((16,), -1.0, jnp.float32)

        cin = [None] * _NBUF
        cout = [None] * _NBUF

        def start_in(g):
            s = g % _NBUF
            cin[s] = pltpu.async_copy(
                in_hbm.at[pl.ds(base + g * _CH, _CH)], bufs[s], in_sems[s]
            )

        for j in range(min(_PF, _NCH)):
            start_in(j)
        for g in range(_NCH):
            s = g % _NBUF
            pf = g + _PF
            if pf < _NCH:
                if pf >= _NBUF:
                    cout[pf % _NBUF].wait()
                start_in(pf)
            cin[s].wait()
            for k in range(_CH // 16):
                rows = lax.iota(jnp.int32, 16) + (k * 16)
                cols = mv_v[pl.ds(g * _CH + k * 16, 16)]
                plsc.store_scatter(bufs[s], [rows, cols], neg1)
            cout[s] = pltpu.async_copy(
                bufs[s], out_hbm.at[pl.ds(base + g * _CH, _CH)], out_sems[s]
            )
        for g in range(_NCH - _NBUF, _NCH):
            cout[g % _NBUF].wait()

    return sc(input, masked_values)


# speedup vs baseline: 7.6726x; 4.2447x over previous
import jax
import jax.numpy as jnp
from jax.experimental import pallas as pl

_BC = 1024


def _mask_body(x_ref, mv_ref, o_ref):
    x = x_ref[...]
    mv = mv_ref[0, 0, :]
    row = jax.lax.broadcasted_iota(jnp.int32, x.shape, 0)
    o_ref[...] = jnp.where(row == mv[None, :], jnp.float32(-1.0), x)


def kernel(input, masked_values):
    B, C = input.shape
    inp_t = input.T
    grid = (B // _BC,)
    mv3 = masked_values.reshape(grid[0], 1, _BC)
    out_t = pl.pallas_call(
        _mask_body,
        grid=grid,
        in_specs=[
            pl.BlockSpec((C, _BC), lambda i: (0, i)),
            pl.BlockSpec((1, 1, _BC), lambda i: (i, 0, 0)),
        ],
        out_specs=pl.BlockSpec((C, _BC), lambda i: (0, i)),
        out_shape=jax.ShapeDtypeStruct((C, B), input.dtype),
    )(inp_t, mv3)
    return out_t.T


# speedup vs baseline: 7.9143x; 1.0315x over previous
import jax
import jax.numpy as jnp
from jax.experimental import pallas as pl

_BC = 2048


def _mask_body(x_ref, mv_ref, o_ref):
    x = x_ref[...]
    mv = mv_ref[0, 0, :]
    row = jax.lax.broadcasted_iota(jnp.int32, x.shape, 0)
    o_ref[...] = jnp.where(row == mv[None, :], jnp.float32(-1.0), x)


def kernel(input, masked_values):
    B, C = input.shape
    inp_t = input.T
    grid = (B // _BC,)
    mv3 = masked_values.reshape(grid[0], 1, _BC)
    out_t = pl.pallas_call(
        _mask_body,
        grid=grid,
        in_specs=[
            pl.BlockSpec((C, _BC), lambda i: (0, i)),
            pl.BlockSpec((1, 1, _BC), lambda i: (i, 0, 0)),
        ],
        out_specs=pl.BlockSpec((C, _BC), lambda i: (0, i)),
        out_shape=jax.ShapeDtypeStruct((C, B), input.dtype),
    )(inp_t, mv3)
    return out_t.T
